# baseline (device time: 31491 ns/iter reference)
import jax
import jax.numpy as jnp
from jax import lax
from jax.experimental import pallas as pl
from jax.experimental.pallas import tpu as pltpu

N_DEV = 4
B = 2
SQ = 256
SKV = 512
D = 768
HQ_LOC = 8
HKV_LOC = 2
DH = 64
SCALE = 0.125


def kernel(x, Wq, Wo, K_ext, V_ext):
    def body(x_ref, wq_ref, wo_hbm, k_hbm, v_hbm, out_ref,
             kv_ref, vv_ref, wo_ref, send_ref, recv_ref,
             send_sems, recv_sems, copy_sems):
        my = lax.axis_index("i")
        p1 = my ^ 1
        p2 = 3 - my
        partners = ((p1, p2), (p2, p1))

        barrier = pltpu.get_barrier_semaphore()
        for p in (p1, p2):
            pl.semaphore_signal(
                barrier, inc=1,
                device_id=(p,), device_id_type=pl.DeviceIdType.MESH,
            )
        pl.semaphore_wait(barrier, 2)

        k_copy = pltpu.make_async_copy(
            k_hbm.at[:, :, pl.ds(2 * my, HKV_LOC), :], kv_ref, copy_sems.at[0])
        v_copy = pltpu.make_async_copy(
            v_hbm.at[:, :, pl.ds(2 * my, HKV_LOC), :], vv_ref, copy_sems.at[1])
        wo_copy = pltpu.make_async_copy(wo_hbm, wo_ref, copy_sems.at[2])
        k_copy.start()
        v_copy.start()
        wo_copy.start()

        wq = (wq_ref[...] * SCALE).astype(jnp.bfloat16)
        x_all = x_ref[...].reshape(B * SQ, D).astype(jnp.bfloat16)
        q_all = jnp.dot(x_all, wq, preferred_element_type=jnp.float32)
        ones_col = jnp.ones((SKV, 1), jnp.bfloat16)

        k_copy.wait()
        v_copy.wait()
        wo_copy.wait()
        wo = wo_ref[...].astype(jnp.bfloat16)

        streams = (
            (0, 0, 640, 0), (0, 640, 128, 1),
            (1, 0, 256, 1), (1, 256, 256, 1),
            (1, 512, 128, 1), (1, 640, 128, 0),
        )

        def start_exchange(r, si, payload_bf16):
            b, c0, clen, par = streams[si]
            send_ref[r, b, :, c0:c0 + clen] = payload_bf16
            order = (p1, p2) if par == 0 else (p2, p1)
            rdma = pltpu.make_async_remote_copy(
                src_ref=send_ref.at[r, b, :, pl.ds(c0, clen)],
                dst_ref=recv_ref.at[r, b, :, pl.ds(c0, clen)],
                send_sem=send_sems.at[r, si],
                recv_sem=recv_sems.at[r, si],
                device_id=(order[r],),
                device_id_type=pl.DeviceIdType.MESH,
            )
            rdma.start()
            return rdma

        def attention(b):
            cols = []
            for g in range(HKV_LOC):
                q4 = jnp.concatenate(
                    [q_all[b * SQ:(b + 1) * SQ,
                           (4 * g + r) * DH:(4 * g + r + 1) * DH]
                     for r in range(4)], axis=0,
                ).astype(jnp.bfloat16)
                kg = kv_ref[b, :, g, :].astype(jnp.bfloat16)
                s = lax.dot_general(
                    q4, kg, (((1,), (1,)), ((), ())),
                    preferred_element_type=jnp.float32,
                )
                e = jnp.exp(s).astype(jnp.bfloat16)
                vg = jnp.concatenate(
                    [vv_ref[b, :, g, :].astype(jnp.bfloat16), ones_col],
                    axis=1)
                ov = jnp.dot(e, vg, preferred_element_type=jnp.float32)
                o4 = ov[:, :DH] / ov[:, DH:DH + 1]
                cols.extend([o4[r * SQ:(r + 1) * SQ, :] for r in range(4)])
            return jnp.concatenate(cols, axis=1).astype(jnp.bfloat16)

        partial = {}
        rd0, rd1 = {}, {}

        def project_and_send(si, attn_b):
            b, c0, clen, _ = streams[si]
            partial[si] = jnp.dot(attn_b, wo[:, c0:c0 + clen],
                                  preferred_element_type=jnp.float32)
            rd0[si] = start_exchange(0, si, partial[si].astype(jnp.bfloat16))

        def finish_r0_send_r1(si):
            b, c0, clen, _ = streams[si]
            rd0[si].wait()
            acc = partial[si] + recv_ref[0, b, :, c0:c0 + clen].astype(
                jnp.float32)
            partial[si] = acc
            rd1[si] = start_exchange(1, si, acc.astype(jnp.bfloat16))

        attn0 = attention(0)
        project_and_send(0, attn0)
        project_and_send(1, attn0)
        attn1 = attention(1)
        finish_r0_send_r1(0)
        finish_r0_send_r1(1)
        for si in (2, 3, 4, 5):
            project_and_send(si, attn1)
        for si in (2, 3, 4, 5):
            finish_r0_send_r1(si)
        for si in range(6):
            b, c0, clen, _ = streams[si]
            rd1[si].wait()
            out_ref[b, :, c0:c0 + clen] = (
                partial[si]
                + recv_ref[1, b, :, c0:c0 + clen].astype(jnp.float32))

    return pl.pallas_call(
        body,
        out_shape=jax.ShapeDtypeStruct((B, SQ, D), jnp.float32),
        in_specs=[
            pl.BlockSpec(memory_space=pltpu.VMEM),
            pl.BlockSpec(memory_space=pltpu.VMEM),
            pl.BlockSpec(memory_space=pl.ANY),
            pl.BlockSpec(memory_space=pl.ANY),
            pl.BlockSpec(memory_space=pl.ANY),
        ],
        out_specs=pl.BlockSpec(memory_space=pltpu.VMEM),
        scratch_shapes=[
            pltpu.VMEM((B, SKV, HKV_LOC, DH), jnp.float32),
            pltpu.VMEM((B, SKV, HKV_LOC, DH), jnp.float32),
            pltpu.VMEM((512, D), jnp.float32),
            pltpu.VMEM((2, B, SQ, D), jnp.bfloat16),
            pltpu.VMEM((2, B, SQ, D), jnp.bfloat16),
            pltpu.SemaphoreType.DMA((2, 6)),
            pltpu.SemaphoreType.DMA((2, 6)),
            pltpu.SemaphoreType.DMA((3,)),
        ],
        compiler_params=pltpu.CompilerParams(collective_id=0),
    )(x, Wq, Wo, K_ext, V_ext)


# device time: 22129 ns/iter; 1.4231x vs baseline; 1.4231x over previous
import jax
import jax.numpy as jnp
from jax import lax
from jax.experimental import pallas as pl
from jax.experimental.pallas import tpu as pltpu

N_DEV = 4
B = 2
SQ = 256
SKV = 512
D = 768
HQ_LOC = 8
HKV_LOC = 2
DH = 64
SCALE = 0.125


def kernel(x, Wq, Wo, K_ext, V_ext):
    my_pos = lax.axis_index("i")
    Kt = jnp.transpose(
        lax.dynamic_slice_in_dim(K_ext, 2 * my_pos, HKV_LOC, axis=2),
        (0, 2, 3, 1)).astype(jnp.bfloat16)
    Vt = jnp.transpose(
        lax.dynamic_slice_in_dim(V_ext, 2 * my_pos, HKV_LOC, axis=2),
        (0, 2, 3, 1)).astype(jnp.bfloat16)
    wqb = (Wq * SCALE).astype(jnp.bfloat16)
    wob = Wo.astype(jnp.bfloat16)

    def body(x_ref, wq_ref, wo_ref, kt_ref, vt_ref, out_ref,
             fin_ref, send_ref, recv_ref, send_sems, recv_sems, copy_sems):
        my = lax.axis_index("i")
        p1 = my ^ 1
        p2 = 3 - my
        partners = ((p1, p2), (p2, p1))

        barrier = pltpu.get_barrier_semaphore()
        for p in (p1, p2):
            pl.semaphore_signal(
                barrier, inc=1,
                device_id=(p,), device_id_type=pl.DeviceIdType.MESH,
            )
        pl.semaphore_wait(barrier, 2)

        wq = wq_ref[...]
        wo = wo_ref[...]
        x_all = x_ref[...].reshape(B * SQ, D).astype(jnp.bfloat16)
        q_all = jnp.dot(x_all, wq,
                        preferred_element_type=jnp.float32
                        ).astype(jnp.bfloat16)
        ones_row = jnp.ones((1, SKV), jnp.bfloat16)

        D2 = D // 2

        def stream_partners(b, j):
            return (p1, p2) if (b + j) % 2 == 0 else (p2, p1)

        def start_exchange(r, b, j, payload_bf16):
            s = 2 * b + j
            send_ref[r, s] = payload_bf16
            rdma = pltpu.make_async_remote_copy(
                src_ref=send_ref.at[r, s],
                dst_ref=recv_ref.at[r, s],
                send_sem=send_sems.at[r, s],
                recv_sem=recv_sems.at[r, s],
                device_id=(stream_partners(b, j)[r],),
                device_id_type=pl.DeviceIdType.MESH,
            )
            rdma.start()
            return rdma

        def attention(b):
            cols = []
            for g in range(HKV_LOC):
                q4 = jnp.concatenate(
                    [q_all[b * SQ:(b + 1) * SQ,
                           (4 * g + r) * DH:(4 * g + r + 1) * DH]
                     for r in range(4)], axis=0,
                )
                kg_t = kt_ref[b, g]
                s = jnp.dot(q4, kg_t,
                            preferred_element_type=jnp.float32)
                e = jnp.exp(s).astype(jnp.bfloat16)
                vg_t = jnp.concatenate([vt_ref[b, g], ones_row],
                                       axis=0)
                ov = lax.dot_general(
                    e, vg_t, (((1,), (1,)), ((), ())),
                    preferred_element_type=jnp.float32)
                o4 = ov[:, :DH] / ov[:, DH:DH + 1]
                cols.extend([o4[r * SQ:(r + 1) * SQ, :] for r in range(4)])
            return jnp.concatenate(cols, axis=1).astype(jnp.bfloat16)

        partial = {}
        rd0, rd1 = {}, {}

        def proj_send(b, j, attn_b):
            p = jnp.dot(attn_b, wo[:, j * D2:(j + 1) * D2],
                        preferred_element_type=jnp.float32)
            partial[(b, j)] = p
            rd0[(b, j)] = start_exchange(0, b, j, p.astype(jnp.bfloat16))

        def finish_r0_send_r1(b, j):
            rd0[(b, j)].wait()
            acc = partial[(b, j)] + recv_ref[0, 2 * b + j].astype(jnp.float32)
            partial[(b, j)] = acc
            rd1[(b, j)] = start_exchange(1, b, j, acc.astype(jnp.bfloat16))

        attn0 = attention(0)
        proj_send(0, 0, attn0)
        proj_send(0, 1, attn0)
        attn1 = attention(1)
        proj_send(1, 0, attn1)
        proj_send(1, 1, attn1)
        finish_r0_send_r1(0, 0)
        finish_r0_send_r1(0, 1)
        finish_r0_send_r1(1, 0)
        finish_r0_send_r1(1, 1)
        out_copies = []
        for b in range(B):
            for j in range(2):
                rd1[(b, j)].wait()
                fin_ref[2 * b + j] = (
                    partial[(b, j)]
                    + recv_ref[1, 2 * b + j].astype(jnp.float32)
                ).astype(jnp.bfloat16)
                oc = pltpu.make_async_copy(
                    fin_ref.at[2 * b + j],
                    out_ref.at[b, :, pl.ds(j * D2, D2)],
                    copy_sems.at[2 * b + j],
                )
                oc.start()
                out_copies.append(oc)
        for oc in out_copies:
            oc.wait()

    return pl.pallas_call(
        body,
        out_shape=jax.ShapeDtypeStruct((B, SQ, D), jnp.bfloat16),
        in_specs=[pl.BlockSpec(memory_space=pltpu.VMEM)] * 5,
        out_specs=pl.BlockSpec(memory_space=pl.ANY),
        scratch_shapes=[
            pltpu.VMEM((2 * B, SQ, D // 2), jnp.bfloat16),
            pltpu.VMEM((2, 2 * B, SQ, D // 2), jnp.bfloat16),
            pltpu.VMEM((2, 2 * B, SQ, D // 2), jnp.bfloat16),
            pltpu.SemaphoreType.DMA((2, 2 * B)),
            pltpu.SemaphoreType.DMA((2, 2 * B)),
            pltpu.SemaphoreType.DMA((2 * B,)),
        ],
        compiler_params=pltpu.CompilerParams(collective_id=0),
    )(x, wqb, wob, Kt, Vt)
